# baseline (speedup 1.0000x reference)
import functools

import jax
import jax.numpy as jnp
from jax import lax
from jax.experimental import pallas as pl
from jax.experimental.pallas import tpu as pltpu
from jax.experimental.pallas import tpu_sc as plsc

N = 10000
E = 320000
NPAD = 10112
EPAD = 327680
RPT = NPAD // 16
CHUNK = 128
NBUF = 2
KAHEAD = NBUF - 1


def _make_spmm(n_tables, dc, t_out, feature_mode, use_table=True):
    passes = t_out // 2
    nchunks = (EPAD // 16 // CHUNK) if feature_mode else (EPAD // 32 // CHUNK)
    schunks = 16
    nstages = nchunks // schunks
    mesh = plsc.VectorSubcoreMesh(core_axis_name="c", subcore_axis_name="s")

    def spmm(*refs):
        if use_table:
            tables = refs[0]
            refs = refs[1:]
        (col2d, row2d, zeros, out, colbuf, rowbuf) = refs[:6]
        gbufs = refs[6:6 + NBUF]
        agg = refs[6 + NBUF]
        gsems = refs[7 + NBUF:7 + 2 * NBUF]
        ssems = refs[7 + 2 * NBUF:7 + 3 * NBUF]
        c = lax.axis_index("c")
        s = lax.axis_index("s")
        wid = s * 2 + c
        cbase = s * nchunks if feature_mode else wid * nchunks

        def gsrc(plane, idx_ref):
            t = plane if feature_mode else 0
            return tables.at[t].at[idx_ref]

        if not use_table:
            def fill(i, _):
                for k in range(dc // 16):
                    gbufs[0][i, pl.ds(k * 16, 16)] = jnp.ones((16,), jnp.float32)
                return 0
            lax.fori_loop(0, CHUNK, fill, 0)
        for p in range(passes):
            if p:
                plsc.subcore_barrier()
            plane = c * passes + p
            pltpu.sync_copy(zeros.at[pl.ds(s * RPT, RPT)],
                            agg.at[pl.ds(s * RPT, RPT)])
            plsc.subcore_barrier()

            for st in range(nstages):
                sbase = cbase + st * schunks
                if use_table:
                    pltpu.sync_copy(col2d.at[pl.ds(sbase, schunks)], colbuf)
                pltpu.sync_copy(row2d.at[pl.ds(sbase, schunks)], rowbuf)

                if not use_table:
                    def dbody(j, _):
                        pltpu.sync_copy(gbufs[0], agg.at[rowbuf.at[j]], add=True)
                        return 0
                    lax.fori_loop(0, schunks, dbody, 0)
                else:
                    for k in range(KAHEAD):
                        pltpu.async_copy(gsrc(plane, colbuf.at[k]),
                                         gbufs[k], gsems[k])

                    def body(j4, _):
                        for b in range(NBUF):
                            j = NBUF * j4 + b
                            pltpu.make_async_copy(
                                gsrc(plane, colbuf.at[j]), gbufs[b],
                                gsems[b]).wait()
                            sw = ssems[(b - 1) % NBUF]

                            def _wait_s(sw=sw, j=j):
                                pltpu.make_async_copy(
                                    gbufs[0], agg.at[rowbuf.at[j]], sw).wait()
                            if b == 0:
                                pl.when(j4 > 0)(_wait_s)
                            else:
                                _wait_s()
                            gi = gbufs[(b + KAHEAD) % NBUF]
                            gs2 = gsems[(b + KAHEAD) % NBUF]

                            def _issue_g(gi=gi, gs2=gs2, j=j):
                                pltpu.async_copy(
                                    gsrc(plane, colbuf.at[j + KAHEAD]), gi, gs2)
                            if b == 0:
                                _issue_g()
                            else:
                                pl.when(j4 < schunks // NBUF - 1)(_issue_g)
                            pltpu.async_copy(
                                gbufs[b], agg.at[rowbuf.at[j]], ssems[b],
                                add=True)
                        return 0

                    lax.fori_loop(0, schunks // NBUF, body, 0)
                    j = schunks - 1
                    pltpu.make_async_copy(
                        gbufs[j % NBUF], agg.at[rowbuf.at[j]],
                        ssems[j % NBUF]).wait()
            plsc.subcore_barrier()
            pltpu.sync_copy(agg.at[pl.ds(s * RPT, RPT)],
                            out.at[plane, pl.ds(s * RPT, RPT)])

    return functools.partial(
        pl.kernel,
        out_type=jax.ShapeDtypeStruct((t_out, NPAD, dc), jnp.float32),
        mesh=mesh,
        scratch_types=(
            [pltpu.VMEM((schunks, CHUNK), jnp.int32),
             pltpu.VMEM((schunks, CHUNK), jnp.int32)]
            + [pltpu.VMEM((CHUNK, dc), jnp.float32)] * NBUF
            + [pltpu.VMEM_SHARED((NPAD, dc), jnp.float32)]
            + [pltpu.SemaphoreType.DMA] * (2 * NBUF)
        ),
    )(spmm)


_spmm_deg = _make_spmm(0, 128, 2, False, use_table=False)
_spmm_l1 = _make_spmm(1, 128, 2, False)
_spmm_l2 = _make_spmm(2, 128, 2, True)
_spmm_l3 = _make_spmm(4, 128, 4, True)


BR = 400


def _prep_body(d0, d1, x, xs1, dinv_rep):
    deg = d0[...][:, 0:1] + d1[...][:, 0:1]
    dinv = jnp.where(deg > 0, lax.rsqrt(deg), 0.0)
    xs1[...] = x[...] * dinv
    dinv_rep[...] = jnp.broadcast_to(dinv, (BR, 128))


def _prep(d0, d1, x):
    return pl.pallas_call(
        _prep_body,
        grid=(N // BR,),
        in_specs=[
            pl.BlockSpec((BR, 16), lambda i: (i, 0)),
            pl.BlockSpec((BR, 16), lambda i: (i, 0)),
            pl.BlockSpec((BR, 128), lambda i: (i, 0)),
        ],
        out_specs=[
            pl.BlockSpec((BR, 128), lambda i: (i, 0)),
            pl.BlockSpec((BR, 128), lambda i: (i, 0)),
        ],
        out_shape=[
            jax.ShapeDtypeStruct((N, 128), jnp.float32),
            jax.ShapeDtypeStruct((N, 128), jnp.float32),
        ],
    )(d0, d1, x)


def _make_pre(din, dout):
    def body(h_in, W0, b, acc):
        acc[...] = (jnp.dot(h_in[...], W0[...],
                            preferred_element_type=jnp.float32) + b[...])

    call = pl.pallas_call(
        body,
        grid=(N // BR,),
        in_specs=[
            pl.BlockSpec((BR, din), lambda i: (i, 0)),
            pl.BlockSpec((din, dout), lambda i: (0, 0)),
            pl.BlockSpec((1, dout), lambda i: (0, 0)),
        ],
        out_specs=pl.BlockSpec((BR, dout), lambda i: (i, 0)),
        out_shape=jax.ShapeDtypeStruct((N, dout), jnp.float32),
    )

    def run(h_in, W0, b):
        return call(h_in, W0, b.reshape(1, dout))

    return run


def _make_combine(dout, t_in, sum_mode, t_next):
    def body(acc, raw, dinv, W1, h_out, *xs_out):
        dv = dinv[...][:, 0:1]
        if sum_mode:
            rawcat = raw[...][0] + raw[...][1]
        else:
            r = raw[...]
            rawcat = jnp.concatenate([r[t] for t in range(t_in)], axis=1)
        tx = -(rawcat * dv)
        h = jnp.maximum(
            acc[...] + jnp.dot(tx, W1[...],
                               preferred_element_type=jnp.float32), 0.0)
        h_out[...] = h
        if t_next:
            xs = (h * dv).reshape(BR, t_next, 128).transpose(1, 0, 2)
            xs_out[0][...] = xs

    din1 = t_in * 128 if not sum_mode else 128
    out_specs = [pl.BlockSpec((BR, dout), lambda i: (i, 0))]
    out_shape = [jax.ShapeDtypeStruct((N, dout), jnp.float32)]
    if t_next:
        out_specs.append(pl.BlockSpec((t_next, BR, 128), lambda i: (0, i, 0)))
        out_shape.append(jax.ShapeDtypeStruct((t_next, N, 128), jnp.float32))

    return pl.pallas_call(
        body,
        grid=(N // BR,),
        in_specs=[
            pl.BlockSpec((BR, dout), lambda i: (i, 0)),
            pl.BlockSpec((t_in, BR, 128), lambda i: (0, i, 0)),
            pl.BlockSpec((BR, 128), lambda i: (i, 0)),
            pl.BlockSpec((din1, dout), lambda i: (0, 0)),
        ],
        out_specs=out_specs,
        out_shape=out_shape,
    )


_pre1 = _make_pre(128, 256)
_pre2 = _make_pre(256, 512)
_pre3 = _make_pre(512, 512)
_combine1 = _make_combine(256, 2, True, 2)
_combine2 = _make_combine(512, 2, False, 4)
_combine3 = _make_combine(512, 4, False, None)


def kernel(x, edge_index, W0_1, W1_1, b1, W0_2, W1_2, b2, W0_3, W1_3, b3):
    row = edge_index[0].astype(jnp.int32)
    col = edge_index[1].astype(jnp.int32)
    pad = EPAD - E
    row2d = jnp.concatenate([row, jnp.full((pad,), N, jnp.int32)]).reshape(
        EPAD // CHUNK, CHUNK)
    col2d = jnp.concatenate([col, jnp.zeros((pad,), jnp.int32)]).reshape(
        EPAD // CHUNK, CHUNK)

    z128 = jnp.zeros((NPAD, 128), jnp.float32)

    deg_pair = _spmm_deg(col2d, row2d, z128)
    acc1 = _pre1(x, W0_1, b1)
    xs1, dinv = _prep(deg_pair[0, :N, :16], deg_pair[1, :N, :16], x)

    raw1 = _spmm_l1(xs1[None], col2d, row2d, z128)
    h1, xs2 = _combine1(acc1, raw1[:, :N], dinv, W1_1)

    acc2 = _pre2(h1, W0_2, b2)
    raw2 = _spmm_l2(xs2, col2d, row2d, z128)
    h2, xs3 = _combine2(acc2, raw2[:, :N], dinv, W1_2)

    acc3 = _pre3(h2, W0_3, b3)
    raw3 = _spmm_l3(xs3, col2d, row2d, z128)
    (h3,) = _combine3(acc3, raw3[:, :N], dinv, W1_3)
    return h3


# speedup vs baseline: 1.1771x; 1.1771x over previous
import functools

import jax
import jax.numpy as jnp
from jax import lax
from jax.experimental import pallas as pl
from jax.experimental.pallas import tpu as pltpu
from jax.experimental.pallas import tpu_sc as plsc

N = 10000
E = 320000
NPAD = 10112
EPAD = 327680
RPT = NPAD // 16
CHUNK = 64
NBUF = 4
KAHEAD = NBUF - 1


def _make_spmm(n_tables, dc, t_out, feature_mode, use_table=True):
    passes = t_out // 2
    nchunks = (EPAD // 16 // CHUNK) if feature_mode else (EPAD // 32 // CHUNK)
    schunks = 40
    nstages = nchunks // schunks
    mesh = plsc.VectorSubcoreMesh(core_axis_name="c", subcore_axis_name="s")

    def spmm(*refs):
        if use_table:
            tables = refs[0]
            refs = refs[1:]
        (col2d, row2d, zeros, out, colbuf, rowbuf) = refs[:6]
        gbufs = refs[6:6 + NBUF]
        agg = refs[6 + NBUF]
        gsems = refs[7 + NBUF:7 + 2 * NBUF]
        ssems = refs[7 + 2 * NBUF:7 + 3 * NBUF]
        c = lax.axis_index("c")
        s = lax.axis_index("s")
        wid = s * 2 + c
        cbase = s * nchunks if feature_mode else wid * nchunks

        def gsrc(plane, idx_ref):
            t = plane if feature_mode else 0
            return tables.at[t].at[idx_ref]

        if not use_table:
            def fill(i, _):
                for k in range(dc // 16):
                    gbufs[0][i, pl.ds(k * 16, 16)] = jnp.ones((16,), jnp.float32)
                return 0
            lax.fori_loop(0, CHUNK, fill, 0)
        for p in range(passes):
            if p:
                plsc.subcore_barrier()
            plane = c * passes + p
            pltpu.sync_copy(zeros.at[pl.ds(s * RPT, RPT)],
                            agg.at[pl.ds(s * RPT, RPT)])
            plsc.subcore_barrier()

            for st in range(nstages):
                sbase = cbase + st * schunks
                if use_table:
                    pltpu.sync_copy(col2d.at[pl.ds(sbase, schunks)], colbuf)
                pltpu.sync_copy(row2d.at[pl.ds(sbase, schunks)], rowbuf)

                if not use_table:
                    def dbody(j, _):
                        pltpu.sync_copy(gbufs[0], agg.at[rowbuf.at[j]], add=True)
                        return 0
                    lax.fori_loop(0, schunks, dbody, 0)
                else:
                    for k in range(KAHEAD):
                        pltpu.async_copy(gsrc(plane, colbuf.at[k]),
                                         gbufs[k], gsems[k])

                    def body(j4, _):
                        for b in range(NBUF):
                            j = NBUF * j4 + b
                            pltpu.make_async_copy(
                                gsrc(plane, colbuf.at[j]), gbufs[b],
                                gsems[b]).wait()
                            sw = ssems[(b - 1) % NBUF]

                            def _wait_s(sw=sw, j=j):
                                pltpu.make_async_copy(
                                    gbufs[0], agg.at[rowbuf.at[j]], sw).wait()
                            if b == 0:
                                pl.when(j4 > 0)(_wait_s)
                            else:
                                _wait_s()
                            gi = gbufs[(b + KAHEAD) % NBUF]
                            gs2 = gsems[(b + KAHEAD) % NBUF]

                            def _issue_g(gi=gi, gs2=gs2, j=j):
                                pltpu.async_copy(
                                    gsrc(plane, colbuf.at[j + KAHEAD]), gi, gs2)
                            if b == 0:
                                _issue_g()
                            else:
                                pl.when(j4 < schunks // NBUF - 1)(_issue_g)
                            pltpu.async_copy(
                                gbufs[b], agg.at[rowbuf.at[j]], ssems[b],
                                add=True)
                        return 0

                    lax.fori_loop(0, schunks // NBUF, body, 0)
                    j = schunks - 1
                    pltpu.make_async_copy(
                        gbufs[j % NBUF], agg.at[rowbuf.at[j]],
                        ssems[j % NBUF]).wait()
            plsc.subcore_barrier()
            pltpu.sync_copy(agg.at[pl.ds(s * RPT, RPT)],
                            out.at[plane, pl.ds(s * RPT, RPT)])

    return functools.partial(
        pl.kernel,
        out_type=jax.ShapeDtypeStruct((t_out, NPAD, dc), jnp.float32),
        mesh=mesh,
        scratch_types=(
            [pltpu.VMEM((schunks, CHUNK), jnp.int32),
             pltpu.VMEM((schunks, CHUNK), jnp.int32)]
            + [pltpu.VMEM((CHUNK, dc), jnp.float32)] * NBUF
            + [pltpu.VMEM_SHARED((NPAD, dc), jnp.float32)]
            + [pltpu.SemaphoreType.DMA] * (2 * NBUF)
        ),
    )(spmm)


_spmm_deg = _make_spmm(0, 128, 2, False, use_table=False)
_spmm_l1 = _make_spmm(1, 128, 2, False)
_spmm_l2 = _make_spmm(2, 128, 2, True)
_spmm_l3 = _make_spmm(4, 128, 4, True)


BR = 400


def _prep_body(d0, d1, x, xs1, dinv_rep):
    deg = d0[...][:, 0:1] + d1[...][:, 0:1]
    dinv = jnp.where(deg > 0, lax.rsqrt(deg), 0.0)
    xs1[...] = x[...] * dinv
    dinv_rep[...] = jnp.broadcast_to(dinv, (BR, 128))


def _prep(d0, d1, x):
    return pl.pallas_call(
        _prep_body,
        grid=(N // BR,),
        in_specs=[
            pl.BlockSpec((BR, 16), lambda i: (i, 0)),
            pl.BlockSpec((BR, 16), lambda i: (i, 0)),
            pl.BlockSpec((BR, 128), lambda i: (i, 0)),
        ],
        out_specs=[
            pl.BlockSpec((BR, 128), lambda i: (i, 0)),
            pl.BlockSpec((BR, 128), lambda i: (i, 0)),
        ],
        out_shape=[
            jax.ShapeDtypeStruct((N, 128), jnp.float32),
            jax.ShapeDtypeStruct((N, 128), jnp.float32),
        ],
    )(d0, d1, x)


def _make_pre(din, dout):
    def body(h_in, W0, b, acc):
        acc[...] = (jnp.dot(h_in[...], W0[...],
                            preferred_element_type=jnp.float32) + b[...])

    call = pl.pallas_call(
        body,
        grid=(N // BR,),
        in_specs=[
            pl.BlockSpec((BR, din), lambda i: (i, 0)),
            pl.BlockSpec((din, dout), lambda i: (0, 0)),
            pl.BlockSpec((1, dout), lambda i: (0, 0)),
        ],
        out_specs=pl.BlockSpec((BR, dout), lambda i: (i, 0)),
        out_shape=jax.ShapeDtypeStruct((N, dout), jnp.float32),
    )

    def run(h_in, W0, b):
        return call(h_in, W0, b.reshape(1, dout))

    return run


def _make_combine(dout, t_in, sum_mode, t_next):
    def body(acc, raw, dinv, W1, h_out, *xs_out):
        dv = dinv[...][:, 0:1]
        if sum_mode:
            rawcat = raw[...][0] + raw[...][1]
        else:
            r = raw[...]
            rawcat = jnp.concatenate([r[t] for t in range(t_in)], axis=1)
        tx = -(rawcat * dv)
        h = jnp.maximum(
            acc[...] + jnp.dot(tx, W1[...],
                               preferred_element_type=jnp.float32), 0.0)
        h_out[...] = h
        if t_next:
            xs = (h * dv).reshape(BR, t_next, 128).transpose(1, 0, 2)
            xs_out[0][...] = xs

    din1 = t_in * 128 if not sum_mode else 128
    out_specs = [pl.BlockSpec((BR, dout), lambda i: (i, 0))]
    out_shape = [jax.ShapeDtypeStruct((N, dout), jnp.float32)]
    if t_next:
        out_specs.append(pl.BlockSpec((t_next, BR, 128), lambda i: (0, i, 0)))
        out_shape.append(jax.ShapeDtypeStruct((t_next, N, 128), jnp.float32))

    return pl.pallas_call(
        body,
        grid=(N // BR,),
        in_specs=[
            pl.BlockSpec((BR, dout), lambda i: (i, 0)),
            pl.BlockSpec((t_in, BR, 128), lambda i: (0, i, 0)),
            pl.BlockSpec((BR, 128), lambda i: (i, 0)),
            pl.BlockSpec((din1, dout), lambda i: (0, 0)),
        ],
        out_specs=out_specs,
        out_shape=out_shape,
    )


_pre1 = _make_pre(128, 256)
_pre2 = _make_pre(256, 512)
_pre3 = _make_pre(512, 512)
_combine1 = _make_combine(256, 2, True, 2)
_combine2 = _make_combine(512, 2, False, 4)
_combine3 = _make_combine(512, 4, False, None)


def kernel(x, edge_index, W0_1, W1_1, b1, W0_2, W1_2, b2, W0_3, W1_3, b3):
    row = edge_index[0].astype(jnp.int32)
    col = edge_index[1].astype(jnp.int32)
    pad = EPAD - E
    trash = N + jnp.arange(pad, dtype=jnp.int32) % (NPAD - N)
    row2d = jnp.concatenate([row, trash]).reshape(EPAD // CHUNK, CHUNK)
    col2d = jnp.concatenate([col, jnp.zeros((pad,), jnp.int32)]).reshape(
        EPAD // CHUNK, CHUNK)

    z128 = jnp.zeros((NPAD, 128), jnp.float32)

    deg_pair = _spmm_deg(col2d, row2d, z128)
    acc1 = _pre1(x, W0_1, b1)
    xs1, dinv = _prep(deg_pair[0, :N, :16], deg_pair[1, :N, :16], x)

    raw1 = _spmm_l1(xs1[None], col2d, row2d, z128)
    h1, xs2 = _combine1(acc1, raw1[:, :N], dinv, W1_1)

    acc2 = _pre2(h1, W0_2, b2)
    raw2 = _spmm_l2(xs2, col2d, row2d, z128)
    h2, xs3 = _combine2(acc2, raw2[:, :N], dinv, W1_2)

    acc3 = _pre3(h2, W0_3, b3)
    raw3 = _spmm_l3(xs3, col2d, row2d, z128)
    (h3,) = _combine3(acc3, raw3[:, :N], dinv, W1_3)
    return h3


# speedup vs baseline: 3.0453x; 2.5871x over previous
import functools

import jax
import jax.numpy as jnp
from jax import lax
from jax.experimental import pallas as pl
from jax.experimental.pallas import tpu as pltpu
from jax.experimental.pallas import tpu_sc as plsc

N = 10000
E = 320000
NPAD = 10112
EPAD = 327680
RPT = NPAD // 16
CHUNK = 64
NBUF = 4
KAHEAD = NBUF - 1


def _make_spmm(n_tables, dc, t_out, feature_mode, use_table=True):
    passes = t_out // 2
    nchunks = (EPAD // 16 // CHUNK) if feature_mode else (EPAD // 32 // CHUNK)
    schunks = 40
    nstages = nchunks // schunks
    mesh = plsc.VectorSubcoreMesh(core_axis_name="c", subcore_axis_name="s")

    def spmm(*refs):
        if use_table:
            tables = refs[0]
            refs = refs[1:]
        (col2d, row2d, zeros, out, colbuf, rowbuf) = refs[:6]
        gbufs = refs[6:6 + NBUF]
        agg = refs[6 + NBUF]
        gsems = refs[7 + NBUF:7 + 2 * NBUF]
        ssems = refs[7 + 2 * NBUF:7 + 3 * NBUF]
        c = lax.axis_index("c")
        s = lax.axis_index("s")
        wid = s * 2 + c
        cbase = s * nchunks if feature_mode else wid * nchunks

        def gsrc(plane, idx_ref):
            t = plane if feature_mode else 0
            return tables.at[t].at[idx_ref]

        if not use_table:
            def fill(i, _):
                for k in range(dc // 16):
                    gbufs[0][i, pl.ds(k * 16, 16)] = jnp.ones((16,), jnp.float32)
                return 0
            lax.fori_loop(0, CHUNK, fill, 0)
        for p in range(passes):
            if p:
                plsc.subcore_barrier()
            plane = c * passes + p
            pltpu.sync_copy(zeros.at[pl.ds(s * RPT, RPT)],
                            agg.at[pl.ds(s * RPT, RPT)])
            plsc.subcore_barrier()

            for st in range(nstages):
                sbase = cbase + st * schunks
                if use_table:
                    pltpu.sync_copy(col2d.at[pl.ds(sbase, schunks)], colbuf)
                pltpu.sync_copy(row2d.at[pl.ds(sbase, schunks)], rowbuf)

                if not use_table:
                    def dbody(j, _):
                        pltpu.sync_copy(gbufs[0], agg.at[rowbuf.at[j]], add=True)
                        return 0
                    lax.fori_loop(0, schunks, dbody, 0)
                else:
                    for k in range(KAHEAD):
                        pltpu.async_copy(gsrc(plane, colbuf.at[k]),
                                         gbufs[k], gsems[k])

                    def body(j4, _):
                        for b in range(NBUF):
                            j = NBUF * j4 + b
                            pltpu.make_async_copy(
                                gsrc(plane, colbuf.at[j]), gbufs[b],
                                gsems[b]).wait()
                            sw = ssems[(b - 1) % NBUF]

                            def _wait_s(sw=sw, j=j):
                                pltpu.make_async_copy(
                                    gbufs[0], agg.at[rowbuf.at[j]], sw).wait()
                            if b == 0:
                                pl.when(j4 > 0)(_wait_s)
                            else:
                                _wait_s()
                            gi = gbufs[(b + KAHEAD) % NBUF]
                            gs2 = gsems[(b + KAHEAD) % NBUF]

                            def _issue_g(gi=gi, gs2=gs2, j=j):
                                pltpu.async_copy(
                                    gsrc(plane, colbuf.at[j + KAHEAD]), gi, gs2)
                            if b == 0:
                                _issue_g()
                            else:
                                pl.when(j4 < schunks // NBUF - 1)(_issue_g)
                            pltpu.async_copy(
                                gbufs[b], agg.at[rowbuf.at[j]], ssems[b],
                                add=True)
                        return 0

                    lax.fori_loop(0, schunks // NBUF, body, 0)
                    j = schunks - 1
                    pltpu.make_async_copy(
                        gbufs[j % NBUF], agg.at[rowbuf.at[j]],
                        ssems[j % NBUF]).wait()
            plsc.subcore_barrier()
            pltpu.sync_copy(agg.at[pl.ds(s * RPT, RPT)],
                            out.at[plane, pl.ds(s * RPT, RPT)])

    return functools.partial(
        pl.kernel,
        out_type=jax.ShapeDtypeStruct((t_out, NPAD, dc), jnp.float32),
        mesh=mesh,
        scratch_types=(
            [pltpu.VMEM((schunks, CHUNK), jnp.int32),
             pltpu.VMEM((schunks, CHUNK), jnp.int32)]
            + [pltpu.VMEM((CHUNK, dc), jnp.float32)] * NBUF
            + [pltpu.VMEM_SHARED((NPAD, dc), jnp.float32)]
            + [pltpu.SemaphoreType.DMA] * (2 * NBUF)
        ),
    )(spmm)


_spmm_deg = _make_spmm(0, 128, 2, False, use_table=False)
_spmm_l1 = _make_spmm(1, 128, 2, False)
_spmm_l2 = _make_spmm(2, 128, 2, True)
_spmm_l3 = _make_spmm(4, 128, 4, True)


BR = 400


def _prep_body(d0, d1, x, xs1, dinv_rep):
    deg = d0[...][:, 0:1] + d1[...][:, 0:1]
    dinv = jnp.where(deg > 0, lax.rsqrt(deg), 0.0)
    xs1[...] = x[...] * dinv
    dinv_rep[...] = jnp.broadcast_to(dinv, (BR, 128))


def _prep(d0, d1, x):
    return pl.pallas_call(
        _prep_body,
        grid=(N // BR,),
        in_specs=[
            pl.BlockSpec((BR, 16), lambda i: (i, 0)),
            pl.BlockSpec((BR, 16), lambda i: (i, 0)),
            pl.BlockSpec((BR, 128), lambda i: (i, 0)),
        ],
        out_specs=[
            pl.BlockSpec((BR, 128), lambda i: (i, 0)),
            pl.BlockSpec((BR, 128), lambda i: (i, 0)),
        ],
        out_shape=[
            jax.ShapeDtypeStruct((N, 128), jnp.float32),
            jax.ShapeDtypeStruct((N, 128), jnp.float32),
        ],
    )(d0, d1, x)


def _make_pre(din, dout):
    def body(h_in, W0, b, acc):
        acc[...] = (jnp.dot(h_in[...], W0[...],
                            preferred_element_type=jnp.float32) + b[...])

    call = pl.pallas_call(
        body,
        grid=(N // BR,),
        in_specs=[
            pl.BlockSpec((BR, din), lambda i: (i, 0)),
            pl.BlockSpec((din, dout), lambda i: (0, 0)),
            pl.BlockSpec((1, dout), lambda i: (0, 0)),
        ],
        out_specs=pl.BlockSpec((BR, dout), lambda i: (i, 0)),
        out_shape=jax.ShapeDtypeStruct((N, dout), jnp.float32),
    )

    def run(h_in, W0, b):
        return call(h_in, W0, b.reshape(1, dout))

    return run


def _make_combine(dout, t_in, sum_mode, t_next):
    def body(acc, raw, dinv, W1, h_out, *xs_out):
        dv = dinv[...][:, 0:1]
        if sum_mode:
            rawcat = raw[...][0] + raw[...][1]
        else:
            r = raw[...]
            rawcat = jnp.concatenate([r[t] for t in range(t_in)], axis=1)
        tx = -(rawcat * dv)
        h = jnp.maximum(
            acc[...] + jnp.dot(tx, W1[...],
                               preferred_element_type=jnp.float32), 0.0)
        h_out[...] = h
        if t_next:
            xs = (h * dv).reshape(BR, t_next, 128).transpose(1, 0, 2)
            xs_out[0][...] = xs

    din1 = t_in * 128 if not sum_mode else 128
    out_specs = [pl.BlockSpec((BR, dout), lambda i: (i, 0))]
    out_shape = [jax.ShapeDtypeStruct((N, dout), jnp.float32)]
    if t_next:
        out_specs.append(pl.BlockSpec((t_next, BR, 128), lambda i: (0, i, 0)))
        out_shape.append(jax.ShapeDtypeStruct((t_next, N, 128), jnp.float32))

    return pl.pallas_call(
        body,
        grid=(N // BR,),
        in_specs=[
            pl.BlockSpec((BR, dout), lambda i: (i, 0)),
            pl.BlockSpec((t_in, BR, 128), lambda i: (0, i, 0)),
            pl.BlockSpec((BR, 128), lambda i: (i, 0)),
            pl.BlockSpec((din1, dout), lambda i: (0, 0)),
        ],
        out_specs=out_specs,
        out_shape=out_shape,
    )


_pre1 = _make_pre(128, 256)
_pre2 = _make_pre(256, 512)
_pre3 = _make_pre(512, 512)
_combine1 = _make_combine(256, 2, True, 2)
_combine2 = _make_combine(512, 2, False, 4)
_combine3 = _make_combine(512, 4, False, None)


def kernel(x, edge_index, W0_1, W1_1, b1, W0_2, W1_2, b2, W0_3, W1_3, b3):
    row = edge_index[0].astype(jnp.int32)
    col = edge_index[1].astype(jnp.int32)
    pad = EPAD - E
    trash = N + jnp.arange(pad, dtype=jnp.int32) % (NPAD - N)
    row2d = jnp.concatenate([row, trash]).reshape(EPAD // CHUNK, CHUNK)
    padcol = jnp.arange(pad, dtype=jnp.int32) * 13 % N
    col2d = jnp.concatenate([col, padcol]).reshape(EPAD // CHUNK, CHUNK)

    z128 = jnp.zeros((NPAD, 128), jnp.float32)

    deg_pair = _spmm_deg(col2d, row2d, z128)
    acc1 = _pre1(x, W0_1, b1)
    xs1, dinv = _prep(deg_pair[0, :N, :16], deg_pair[1, :N, :16], x)

    raw1 = _spmm_l1(xs1[None], col2d, row2d, z128)
    h1, xs2 = _combine1(acc1, raw1[:, :N], dinv, W1_1)

    acc2 = _pre2(h1, W0_2, b2)
    raw2 = _spmm_l2(xs2, col2d, row2d, z128)
    h2, xs3 = _combine2(acc2, raw2[:, :N], dinv, W1_2)

    acc3 = _pre3(h2, W0_3, b3)
    raw3 = _spmm_l3(xs3, col2d, row2d, z128)
    (h3,) = _combine3(acc3, raw3[:, :N], dinv, W1_3)
    return h3


# speedup vs baseline: 3.0542x; 1.0029x over previous
import functools

import jax
import jax.numpy as jnp
from jax import lax
from jax.experimental import pallas as pl
from jax.experimental.pallas import tpu as pltpu
from jax.experimental.pallas import tpu_sc as plsc

N = 10000
E = 320000
NPAD = 10112
EPAD = 327680
RPT = NPAD // 16
CHUNK = 64
NBUF = 4
KAHEAD = NBUF - 1


def _make_spmm(n_tables, dc, t_out, feature_mode, use_table=True):
    passes = t_out // 2
    nchunks = (EPAD // 16 // CHUNK) if feature_mode else (EPAD // 32 // CHUNK)
    schunks = 40
    nstages = nchunks // schunks
    mesh = plsc.VectorSubcoreMesh(core_axis_name="c", subcore_axis_name="s")

    def spmm(*refs):
        if use_table:
            tables = refs[0]
            refs = refs[1:]
        (col2d, row2d, zeros, out, colbuf, rowbuf) = refs[:6]
        gbufs = refs[6:6 + NBUF]
        agg = refs[6 + NBUF]
        gsems = refs[7 + NBUF:7 + 2 * NBUF]
        ssems = refs[7 + 2 * NBUF:7 + 3 * NBUF]
        c = lax.axis_index("c")
        s = lax.axis_index("s")
        wid = s * 2 + c
        cbase = s * nchunks if feature_mode else wid * nchunks

        def gsrc(plane, idx_ref):
            t = plane if feature_mode else 0
            return tables.at[t].at[idx_ref]

        if not use_table:
            def fill(i, _):
                for k in range(dc // 16):
                    gbufs[0][i, pl.ds(k * 16, 16)] = jnp.ones((16,), jnp.float32)
                return 0
            lax.fori_loop(0, CHUNK, fill, 0)
        for p in range(passes):
            if p:
                plsc.subcore_barrier()
            plane = c * passes + p
            pltpu.sync_copy(zeros.at[pl.ds(s * RPT, RPT)],
                            agg.at[pl.ds(s * RPT, RPT)])
            plsc.subcore_barrier()

            for st in range(nstages):
                sbase = cbase + st * schunks
                if use_table:
                    pltpu.sync_copy(col2d.at[pl.ds(sbase, schunks)], colbuf)
                pltpu.sync_copy(row2d.at[pl.ds(sbase, schunks)], rowbuf)

                if not use_table:
                    def dbody(j4, _):
                        for b in range(NBUF):
                            j = NBUF * j4 + b

                            def _wait_d(b=b, j=j):
                                pltpu.make_async_copy(
                                    gbufs[0], agg.at[rowbuf.at[j]],
                                    ssems[b]).wait()
                            pl.when(j4 > 0)(_wait_d)
                            pltpu.async_copy(
                                gbufs[0], agg.at[rowbuf.at[j]], ssems[b],
                                add=True)
                        return 0
                    lax.fori_loop(0, schunks // NBUF, dbody, 0)
                    for b in range(NBUF):
                        j = schunks - NBUF + b
                        pltpu.make_async_copy(
                            gbufs[0], agg.at[rowbuf.at[j]], ssems[b]).wait()
                else:
                    for k in range(KAHEAD):
                        pltpu.async_copy(gsrc(plane, colbuf.at[k]),
                                         gbufs[k], gsems[k])

                    def body(j4, _):
                        for b in range(NBUF):
                            j = NBUF * j4 + b
                            pltpu.make_async_copy(
                                gsrc(plane, colbuf.at[j]), gbufs[b],
                                gsems[b]).wait()
                            sw = ssems[(b - 1) % NBUF]

                            def _wait_s(sw=sw, j=j):
                                pltpu.make_async_copy(
                                    gbufs[0], agg.at[rowbuf.at[j]], sw).wait()
                            if b == 0:
                                pl.when(j4 > 0)(_wait_s)
                            else:
                                _wait_s()
                            gi = gbufs[(b + KAHEAD) % NBUF]
                            gs2 = gsems[(b + KAHEAD) % NBUF]

                            def _issue_g(gi=gi, gs2=gs2, j=j):
                                pltpu.async_copy(
                                    gsrc(plane, colbuf.at[j + KAHEAD]), gi, gs2)
                            if b == 0:
                                _issue_g()
                            else:
                                pl.when(j4 < schunks // NBUF - 1)(_issue_g)
                            pltpu.async_copy(
                                gbufs[b], agg.at[rowbuf.at[j]], ssems[b],
                                add=True)
                        return 0

                    lax.fori_loop(0, schunks // NBUF, body, 0)
                    j = schunks - 1
                    pltpu.make_async_copy(
                        gbufs[j % NBUF], agg.at[rowbuf.at[j]],
                        ssems[j % NBUF]).wait()
            plsc.subcore_barrier()
            pltpu.sync_copy(agg.at[pl.ds(s * RPT, RPT)],
                            out.at[plane, pl.ds(s * RPT, RPT)])

    return functools.partial(
        pl.kernel,
        out_type=jax.ShapeDtypeStruct((t_out, NPAD, dc), jnp.float32),
        mesh=mesh,
        scratch_types=(
            [pltpu.VMEM((schunks, CHUNK), jnp.int32),
             pltpu.VMEM((schunks, CHUNK), jnp.int32)]
            + [pltpu.VMEM((CHUNK, dc), jnp.float32)] * NBUF
            + [pltpu.VMEM_SHARED((NPAD, dc), jnp.float32)]
            + [pltpu.SemaphoreType.DMA] * (2 * NBUF)
        ),
    )(spmm)


_spmm_deg = _make_spmm(0, 128, 2, False, use_table=False)
_spmm_l1 = _make_spmm(1, 128, 2, False)
_spmm_l2 = _make_spmm(2, 128, 2, True)
_spmm_l3 = _make_spmm(4, 128, 4, True)


BR = 400


def _prep_body(d0, d1, x, xs1, dinv_rep):
    deg = d0[...][:, 0:1] + d1[...][:, 0:1]
    dinv = jnp.where(deg > 0, lax.rsqrt(deg), 0.0)
    xs1[...] = x[...] * dinv
    dinv_rep[...] = jnp.broadcast_to(dinv, (BR, 128))


def _prep(d0, d1, x):
    return pl.pallas_call(
        _prep_body,
        grid=(N // BR,),
        in_specs=[
            pl.BlockSpec((BR, 16), lambda i: (i, 0)),
            pl.BlockSpec((BR, 16), lambda i: (i, 0)),
            pl.BlockSpec((BR, 128), lambda i: (i, 0)),
        ],
        out_specs=[
            pl.BlockSpec((BR, 128), lambda i: (i, 0)),
            pl.BlockSpec((BR, 128), lambda i: (i, 0)),
        ],
        out_shape=[
            jax.ShapeDtypeStruct((N, 128), jnp.float32),
            jax.ShapeDtypeStruct((N, 128), jnp.float32),
        ],
    )(d0, d1, x)


def _make_pre(din, dout):
    def body(h_in, W0, b, acc):
        acc[...] = (jnp.dot(h_in[...], W0[...],
                            preferred_element_type=jnp.float32) + b[...])

    call = pl.pallas_call(
        body,
        grid=(N // BR,),
        in_specs=[
            pl.BlockSpec((BR, din), lambda i: (i, 0)),
            pl.BlockSpec((din, dout), lambda i: (0, 0)),
            pl.BlockSpec((1, dout), lambda i: (0, 0)),
        ],
        out_specs=pl.BlockSpec((BR, dout), lambda i: (i, 0)),
        out_shape=jax.ShapeDtypeStruct((N, dout), jnp.float32),
    )

    def run(h_in, W0, b):
        return call(h_in, W0, b.reshape(1, dout))

    return run


def _make_combine(dout, t_in, sum_mode, t_next):
    def body(acc, raw, dinv, W1, h_out, *xs_out):
        dv = dinv[...][:, 0:1]
        if sum_mode:
            rawcat = raw[...][0] + raw[...][1]
        else:
            r = raw[...]
            rawcat = jnp.concatenate([r[t] for t in range(t_in)], axis=1)
        tx = -(rawcat * dv)
        h = jnp.maximum(
            acc[...] + jnp.dot(tx, W1[...],
                               preferred_element_type=jnp.float32), 0.0)
        h_out[...] = h
        if t_next:
            xs = (h * dv).reshape(BR, t_next, 128).transpose(1, 0, 2)
            xs_out[0][...] = xs

    din1 = t_in * 128 if not sum_mode else 128
    out_specs = [pl.BlockSpec((BR, dout), lambda i: (i, 0))]
    out_shape = [jax.ShapeDtypeStruct((N, dout), jnp.float32)]
    if t_next:
        out_specs.append(pl.BlockSpec((t_next, BR, 128), lambda i: (0, i, 0)))
        out_shape.append(jax.ShapeDtypeStruct((t_next, N, 128), jnp.float32))

    return pl.pallas_call(
        body,
        grid=(N // BR,),
        in_specs=[
            pl.BlockSpec((BR, dout), lambda i: (i, 0)),
            pl.BlockSpec((t_in, BR, 128), lambda i: (0, i, 0)),
            pl.BlockSpec((BR, 128), lambda i: (i, 0)),
            pl.BlockSpec((din1, dout), lambda i: (0, 0)),
        ],
        out_specs=out_specs,
        out_shape=out_shape,
    )


_pre1 = _make_pre(128, 256)
_pre2 = _make_pre(256, 512)
_pre3 = _make_pre(512, 512)
_combine1 = _make_combine(256, 2, True, 2)
_combine2 = _make_combine(512, 2, False, 4)
_combine3 = _make_combine(512, 4, False, None)


def kernel(x, edge_index, W0_1, W1_1, b1, W0_2, W1_2, b2, W0_3, W1_3, b3):
    row = edge_index[0].astype(jnp.int32)
    col = edge_index[1].astype(jnp.int32)
    pad = EPAD - E
    trash = N + jnp.arange(pad, dtype=jnp.int32) % (NPAD - N)
    row2d = jnp.concatenate([row, trash]).reshape(EPAD // CHUNK, CHUNK)
    padcol = jnp.arange(pad, dtype=jnp.int32) * 13 % N
    col2d = jnp.concatenate([col, padcol]).reshape(EPAD // CHUNK, CHUNK)

    z128 = jnp.zeros((NPAD, 128), jnp.float32)

    deg_pair = _spmm_deg(col2d, row2d, z128)
    acc1 = _pre1(x, W0_1, b1)
    xs1, dinv = _prep(deg_pair[0, :N, :16], deg_pair[1, :N, :16], x)

    raw1 = _spmm_l1(xs1[None], col2d, row2d, z128)
    h1, xs2 = _combine1(acc1, raw1[:, :N], dinv, W1_1)

    acc2 = _pre2(h1, W0_2, b2)
    raw2 = _spmm_l2(xs2, col2d, row2d, z128)
    h2, xs3 = _combine2(acc2, raw2[:, :N], dinv, W1_2)

    acc3 = _pre3(h2, W0_3, b3)
    raw3 = _spmm_l3(xs3, col2d, row2d, z128)
    (h3,) = _combine3(acc3, raw3[:, :N], dinv, W1_3)
    return h3
